# baseline (speedup 1.0000x reference)
import functools

import jax
import jax.numpy as jnp
from jax import lax
from jax.experimental import pallas as pl
from jax.experimental.pallas import tpu as pltpu
from jax.experimental.pallas import tpu_sc as plsc

_L = 16


def _make_sc_partials(num_img, vocab, per_img):
    info = plsc.get_sparse_core_info()
    nc, ns = info.num_cores, info.num_subcores
    nw = nc * ns
    assert num_img % nw == 0
    rows_per_tile = num_img // nw
    ng = -(-per_img // _L)
    row_pad = ng * _L

    mesh = plsc.VectorSubcoreMesh(core_axis_name="c", subcore_axis_name="s")

    @functools.partial(
        pl.kernel,
        out_type=jax.ShapeDtypeStruct((nw, _L), jnp.float32),
        mesh=mesh,
        compiler_params=pltpu.CompilerParams(needs_layout_passes=False),
        scratch_types=[
            pltpu.VMEM((rows_per_tile * per_img,), jnp.int32),
            pltpu.VMEM((rows_per_tile, row_pad), jnp.int32),
            pltpu.VMEM((rows_per_tile, row_pad), jnp.float32),
            pltpu.VMEM((_L,), jnp.float32),
            pltpu.SemaphoreType.DMA,
        ],
    )
    def sc_kernel(in_hbm, tgt_hbm, out_hbm, tgt_v, idx_v, vals_v, stage_v, sem):
        cid = lax.axis_index("c")
        sid = lax.axis_index("s")
        wid = sid * nc + cid
        base_row = wid * rows_per_tile

        pltpu.sync_copy(
            tgt_hbm.at[pl.ds(base_row * per_img, rows_per_tile * per_img)], tgt_v
        )

        iota = lax.iota(jnp.int32, _L)
        rotlane = [jnp.bitwise_and(iota + rot, _L - 1) for rot in range(_L)]
        earlier = [rotlane[rot] < iota for rot in range(_L)]
        gvalid = [iota + g * _L < per_img for g in range(ng)]

        for r in range(rows_per_tile):
            vbase = (base_row + r) * vocab
            for g in range(ng):
                posc = jnp.minimum(iota + g * _L, per_img - 1) + r * per_img
                w = plsc.load_gather(tgt_v, [posc])
                idx_v[r, pl.ds(g * _L, _L)] = jnp.where(gvalid[g], w, 0) + vbase

        copies = [
            pltpu.async_copy(in_hbm.at[idx_v.at[r]], vals_v.at[r], sem)
            for r in range(rows_per_tile)
        ]

        w_masks = []
        for r in range(rows_per_tile):
            groups = []
            for g in range(ng):
                posc = jnp.minimum(iota + g * _L, per_img - 1) + r * per_img
                gv = plsc.load_gather(tgt_v, [posc])
                groups.append(jnp.where(gvalid[g], gv, -1))
            dup = [None] * ng
            for b in range(ng):
                for rot in range(_L):
                    if rot == 0:
                        rv = groups[b]
                    else:
                        sp = rotlane[rot] + b * _L
                        spc = jnp.minimum(sp, per_img - 1) + r * per_img
                        rv = jnp.where(sp < per_img, plsc.load_gather(tgt_v, [spc]), -1)
                        hit = (groups[b] == rv) & earlier[rot]
                        dup[b] = hit if dup[b] is None else (dup[b] | hit)
                    for g in range(b + 1, ng):
                        hit = groups[g] == rv
                        dup[g] = hit if dup[g] is None else (dup[g] | hit)
            w_masks.append(
                [jnp.logical_not(dup[g]) & gvalid[g] for g in range(ng)]
            )

        for c in copies:
            c.wait()

        acc_a = jnp.float32(0.0)
        acc_b = jnp.float32(0.0)
        acc_m = jnp.int32(0)
        pad_lane = iota == (per_img - (ng - 1) * _L)
        for r in range(rows_per_tile):
            sumv = None
            lenv = None
            for g in range(ng):
                v = vals_v[r, pl.ds(g * _L, _L)]
                sv = jnp.where(w_masks[r][g], v, 0.0)
                lv = jnp.where(w_masks[r][g], 1, 0)
                sumv = sv if sumv is None else sumv + sv
                lenv = lv if lenv is None else lenv + lv
            uniq = jnp.sum(sumv)
            ln = jnp.sum(lenv)
            in0 = jnp.sum(
                jnp.where(pad_lane, vals_v[r, pl.ds((ng - 1) * _L, _L)], 0.0)
            )
            acc_a += uniq - ln.astype(jnp.float32) * in0
            acc_b += in0
            acc_m = jnp.maximum(acc_m, ln)

        part = (
            jnp.where(iota == 0, acc_a, 0.0)
            + jnp.where(iota == 1, acc_b, 0.0)
            + jnp.where(iota == 2, acc_m.astype(jnp.float32), 0.0)
        )
        stage_v[...] = part
        pltpu.sync_copy(stage_v, out_hbm.at[wid])

    return sc_kernel


def _combine(parts, num_img):
    def body(p_ref, o_ref):
        x = p_ref[...]
        a = jnp.sum(x[:, 0])
        b = jnp.sum(x[:, 1])
        m = jnp.max(x[:, 2])
        o_ref[...] = jnp.broadcast_to(
            -(a + m * b) / (jnp.float32(num_img) * m), (1, 1)
        )

    out = pl.pallas_call(
        body, out_shape=jax.ShapeDtypeStruct((1, 1), jnp.float32)
    )(parts)
    return out[0, 0]


@jax.jit
def kernel(input, target):
    num_img, vocab = input.shape
    per_img = (target.shape[0] // num_img) * target.shape[1]
    in_flat = input.reshape(-1)
    tgt_flat = target.reshape(-1).astype(jnp.int32)
    sc = _make_sc_partials(num_img, vocab, per_img)
    parts = sc(in_flat, tgt_flat)
    return _combine(parts, num_img)


# speedup vs baseline: 1.0195x; 1.0195x over previous
import functools

import jax
import jax.numpy as jnp
from jax import lax
from jax.experimental import pallas as pl
from jax.experimental.pallas import tpu as pltpu
from jax.experimental.pallas import tpu_sc as plsc

_L = 16


def _make_sc_gather(num_img, vocab, per_img, row_pad):
    info = plsc.get_sparse_core_info()
    nc, ns = info.num_cores, info.num_subcores
    nw = nc * ns
    assert num_img % nw == 0
    rows_per_tile = num_img // nw
    ng = row_pad // _L

    mesh = plsc.VectorSubcoreMesh(core_axis_name="c", subcore_axis_name="s")

    @functools.partial(
        pl.kernel,
        out_type=jax.ShapeDtypeStruct((num_img, row_pad), jnp.float32),
        mesh=mesh,
        compiler_params=pltpu.CompilerParams(needs_layout_passes=False),
        scratch_types=[
            pltpu.VMEM((rows_per_tile, row_pad), jnp.int32),
            pltpu.VMEM((rows_per_tile, row_pad), jnp.float32),
            pltpu.SemaphoreType.DMA,
        ],
    )
    def sc_kernel(in_hbm, tgt_hbm, out_hbm, idx_v, vals_v, sem):
        cid = lax.axis_index("c")
        sid = lax.axis_index("s")
        wid = sid * nc + cid
        base_row = wid * rows_per_tile

        stage = [
            pltpu.async_copy(
                tgt_hbm.at[base_row + r], idx_v.at[r, pl.ds(0, per_img)], sem
            )
            for r in range(rows_per_tile)
        ]
        for c in stage:
            c.wait()

        iota = lax.iota(jnp.int32, _L)
        for r in range(rows_per_tile):
            vbase = (base_row + r) * vocab
            for g in range(ng):
                v = idx_v[r, pl.ds(g * _L, _L)]
                if (g + 1) * _L > per_img:
                    v = jnp.where(iota + g * _L < per_img, v, 0)
                idx_v[r, pl.ds(g * _L, _L)] = v + vbase

        copies = [
            pltpu.async_copy(in_hbm.at[idx_v.at[r]], vals_v.at[r], sem)
            for r in range(rows_per_tile)
        ]
        for c in copies:
            c.wait()

        pltpu.sync_copy(vals_v, out_hbm.at[pl.ds(base_row, rows_per_tile)])

    return sc_kernel


def _make_tc_reduce(num_img, per_img, row_pad):
    def body(t_ref, v_ref, o_ref):
        t = t_ref[...]
        lane = lax.broadcasted_iota(jnp.int32, (num_img, row_pad), 1)
        dup = None
        for s in range(1, per_img):
            hit = (t == pltpu.roll(t, s, axis=1)) & (lane >= s)
            dup = hit if dup is None else dup | hit
        w = jnp.logical_not(dup) & (lane < per_img)

        vals = v_ref[...]
        uniq = jnp.sum(jnp.where(w, vals, 0.0))
        lens = jnp.sum(w.astype(jnp.float32), axis=1, keepdims=True)
        m = jnp.max(lens)
        in0 = vals[:, per_img : per_img + 1]
        corr = jnp.sum((m - lens) * in0)
        o_ref[...] = jnp.broadcast_to(
            -(uniq + corr) / (jnp.float32(num_img) * m), (1, 1)
        )

    return body


@jax.jit
def kernel(input, target):
    num_img, vocab = input.shape
    per_img = (target.shape[0] // num_img) * target.shape[1]
    row_pad = 128
    tgt = target.reshape(num_img, per_img).astype(jnp.int32)

    sc = _make_sc_gather(num_img, vocab, per_img, row_pad)
    vals = sc(input.reshape(-1), tgt)

    pad = -1 - lax.broadcasted_iota(jnp.int32, (num_img, row_pad - per_img), 1)
    tpad = jnp.concatenate([tgt, pad], axis=1)

    out = pl.pallas_call(
        _make_tc_reduce(num_img, per_img, row_pad),
        out_shape=jax.ShapeDtypeStruct((1, 1), jnp.float32),
    )(tpad, vals)
    return out[0, 0]
